# baseline (speedup 1.0000x reference)
import functools

import jax
import jax.numpy as jnp
from jax import lax
from jax.experimental import pallas as pl
from jax.experimental.pallas import tpu as pltpu
from jax.experimental.pallas import tpu_sc as plsc

N_TOK = 16384
D = 256
K = 1024
BN = 4096
NB = N_TOK // BN

SC_NC = 2
SC_NS = 16
SC_NW = SC_NC * SC_NS
B_PER_W = N_TOK // SC_NW
CH = 64
NCH = B_PER_W // CH
NBUF = 3


def _dist_body(x_ref, emb_ref, idx_ref, loss_ref, emb2_s, e2_s):
    emb = emb_ref[...]
    emb2_s[...] = emb + emb
    e2_s[...] = jnp.sum(emb * emb, axis=0, keepdims=True)

    x = x_ref[...]
    cross2 = jnp.dot(x, emb2_s[...], preferred_element_type=jnp.float32)
    z2 = jnp.sum(x * x, axis=1, keepdims=True)
    e2 = e2_s[...]
    LC = 128
    z2b = jnp.broadcast_to(z2, (BN, LC))
    col0 = lax.broadcasted_iota(jnp.int32, (BN, LC), 1).astype(jnp.float32)
    rv = None
    ri = col0
    for t in range(K // LC):
        v = (z2b + e2[:, t * LC:(t + 1) * LC]) - cross2[:, t * LC:(t + 1) * LC]
        if t == 0:
            rv = v
        else:
            lt = v < rv
            ri = jnp.where(lt, col0 + float(t * LC), ri)
            rv = jnp.minimum(rv, v)
    rvT = rv.T
    riT = ri.T
    mT = jnp.min(rvT, axis=0, keepdims=True)
    idxf = jnp.min(jnp.where(rvT == mT, riT, float(K)), axis=0)
    idx_ref[0, 0, :] = idxf.astype(jnp.int32)

    loss_ref[...] = jnp.broadcast_to(jnp.sum(mT), (1, 8, 128))


_dist_call = pl.pallas_call(
    _dist_body,
    grid=(NB,),
    in_specs=[
        pl.BlockSpec((BN, D), lambda i: (i, 0)),
        pl.BlockSpec((D, K), lambda i: (0, 0)),
    ],
    out_specs=[
        pl.BlockSpec((1, 1, BN), lambda i: (i, 0, 0)),
        pl.BlockSpec((1, 8, 128), lambda i: (i, 0, 0)),
    ],
    out_shape=[
        jax.ShapeDtypeStruct((NB, 1, BN), jnp.int32),
        jax.ShapeDtypeStruct((NB, 8, 128), jnp.float32),
    ],
    scratch_shapes=[
        pltpu.VMEM((D, K), jnp.float32),
        pltpu.VMEM((1, K), jnp.float32),
    ],
    compiler_params=pltpu.CompilerParams(
        dimension_semantics=("parallel",),
    ),
)


def _prep_body(emb_ref, embt_ref):
    embt_ref[...] = emb_ref[...].T


_prep_call = pl.pallas_call(
    _prep_body,
    out_shape=jax.ShapeDtypeStruct((K, D), jnp.float32),
)


def _gather_body(table_hbm, idx_hbm, out_hbm, idx_v, *rest):
    rows = rest[:NBUF]
    gsem = rest[NBUF:2 * NBUF]
    ssem = rest[2 * NBUF:]
    wid = lax.axis_index("s") * SC_NC + lax.axis_index("c")
    base = wid * B_PER_W
    pltpu.sync_copy(idx_hbm.at[wid], idx_v)
    gathers = [None] * NCH
    stores = [None] * NCH
    for c in range(NBUF):
        gathers[c] = pltpu.async_copy(
            table_hbm.at[idx_v.at[c]], rows[c], gsem[c])
    for c in range(NCH):
        b = c % NBUF
        gathers[c].wait()
        stores[c] = pltpu.async_copy(
            rows[b], out_hbm.at[pl.ds(base + c * CH, CH)], ssem[b])
        if c + NBUF < NCH:
            stores[c].wait()
            gathers[c + NBUF] = pltpu.async_copy(
                table_hbm.at[idx_v.at[c + NBUF]], rows[b], gsem[b])
    for c in range(NCH - NBUF, NCH):
        stores[c].wait()


@functools.cache
def _gather_call():
    return functools.partial(
        pl.kernel,
        out_type=jax.ShapeDtypeStruct((N_TOK, D), jnp.float32),
        mesh=plsc.VectorSubcoreMesh(
            core_axis_name="c", subcore_axis_name="s",
            num_cores=SC_NC, num_subcores=SC_NS,
        ),
        scratch_types=(
            [pltpu.VMEM((NCH, CH), jnp.int32)]
            + [pltpu.VMEM((CH, D), jnp.float32)] * NBUF
            + [pltpu.SemaphoreType.DMA] * (2 * NBUF)
        ),
    )(_gather_body)


def kernel(_inputs, embeddings):
    x = _inputs.reshape(N_TOK, D)
    emb_t = _prep_call(embeddings)
    idx3, loss_parts = _dist_call(x, embeddings)
    e_k = _gather_call()(emb_t, idx3.reshape(SC_NW, NCH, CH))
    loss = jnp.sum(loss_parts[:, 0, 0]) * (1.25 / (N_TOK * D))
    return e_k.reshape(_inputs.shape), loss


# speedup vs baseline: 1.0544x; 1.0544x over previous
import functools

import jax
import jax.numpy as jnp
from jax import lax
from jax.experimental import pallas as pl
from jax.experimental.pallas import tpu as pltpu
from jax.experimental.pallas import tpu_sc as plsc

N_TOK = 16384
D = 256
K = 1024
BN = 4096
NB = N_TOK // BN

SC_NC = 2
SC_NS = 16
SC_NW = SC_NC * SC_NS
B_PER_W = N_TOK // SC_NW
CH = 64
NCH = B_PER_W // CH
NBUF = 3


def _dist_body(x_ref, emb_ref, idx_ref, loss_ref, embt_ref, emb2_s, e2_s):
    i = pl.program_id(0)

    @pl.when(i == 0)
    def _():
        emb = emb_ref[...]
        emb2_s[...] = emb + emb
        e2_s[...] = jnp.sum(emb * emb, axis=0, keepdims=True)
        embt_ref[...] = emb.T
        loss_ref[0, 0] = 0.0

    x = x_ref[...]
    cross2 = jnp.dot(x, emb2_s[...], preferred_element_type=jnp.float32)
    z2 = jnp.sum(x * x, axis=1, keepdims=True)
    e2 = e2_s[...]
    LC = 128
    z2b = jnp.broadcast_to(z2, (BN, LC))
    col0 = lax.broadcasted_iota(jnp.int32, (BN, LC), 1).astype(jnp.float32)
    rv = None
    ri = col0
    for t in range(K // LC):
        v = (z2b + e2[:, t * LC:(t + 1) * LC]) - cross2[:, t * LC:(t + 1) * LC]
        if t == 0:
            rv = v
        else:
            lt = v < rv
            ri = jnp.where(lt, col0 + float(t * LC), ri)
            rv = jnp.minimum(rv, v)
    rvT = rv.T
    riT = ri.T
    mT = jnp.min(rvT, axis=0, keepdims=True)
    idxf = jnp.min(jnp.where(rvT == mT, riT, float(K)), axis=0)
    idx_ref[0, 0, :] = idxf.astype(jnp.int32)

    loss_ref[0, 0] += jnp.sum(mT)

    @pl.when(i == NB - 1)
    def _():
        loss_ref[0, 0] *= 1.25 / (N_TOK * D)


_dist_call = pl.pallas_call(
    _dist_body,
    grid=(NB,),
    in_specs=[
        pl.BlockSpec((BN, D), lambda i: (i, 0)),
        pl.BlockSpec((D, K), lambda i: (0, 0)),
    ],
    out_specs=[
        pl.BlockSpec((1, 1, BN), lambda i: (i, 0, 0)),
        pl.BlockSpec((1, 1), lambda i: (0, 0), memory_space=pltpu.SMEM),
        pl.BlockSpec((K, D), lambda i: (0, 0)),
    ],
    out_shape=[
        jax.ShapeDtypeStruct((NB, 1, BN), jnp.int32),
        jax.ShapeDtypeStruct((1, 1), jnp.float32),
        jax.ShapeDtypeStruct((K, D), jnp.float32),
    ],
    scratch_shapes=[
        pltpu.VMEM((D, K), jnp.float32),
        pltpu.VMEM((1, K), jnp.float32),
    ],
    compiler_params=pltpu.CompilerParams(
        dimension_semantics=("arbitrary",),
    ),
)


def _gather_body(table_hbm, idx_hbm, out_hbm, idx_v, *rest):
    rows = rest[:NBUF]
    gsem = rest[NBUF:2 * NBUF]
    ssem = rest[2 * NBUF:]
    wid = lax.axis_index("s") * SC_NC + lax.axis_index("c")
    base = wid * B_PER_W
    pltpu.sync_copy(idx_hbm.at[wid], idx_v)
    gathers = [None] * NCH
    stores = [None] * NCH
    for c in range(NBUF):
        gathers[c] = pltpu.async_copy(
            table_hbm.at[idx_v.at[c]], rows[c], gsem[c])
    for c in range(NCH):
        b = c % NBUF
        gathers[c].wait()
        stores[c] = pltpu.async_copy(
            rows[b], out_hbm.at[pl.ds(base + c * CH, CH)], ssem[b])
        if c + NBUF < NCH:
            stores[c].wait()
            gathers[c + NBUF] = pltpu.async_copy(
                table_hbm.at[idx_v.at[c + NBUF]], rows[b], gsem[b])
    for c in range(NCH - NBUF, NCH):
        stores[c].wait()


@functools.cache
def _gather_call():
    return functools.partial(
        pl.kernel,
        out_type=jax.ShapeDtypeStruct((N_TOK, D), jnp.float32),
        mesh=plsc.VectorSubcoreMesh(
            core_axis_name="c", subcore_axis_name="s",
            num_cores=SC_NC, num_subcores=SC_NS,
        ),
        scratch_types=(
            [pltpu.VMEM((NCH, CH), jnp.int32)]
            + [pltpu.VMEM((CH, D), jnp.float32)] * NBUF
            + [pltpu.SemaphoreType.DMA] * (2 * NBUF)
        ),
    )(_gather_body)


def kernel(_inputs, embeddings):
    x = _inputs.reshape(N_TOK, D)
    idx3, loss, emb_t = _dist_call(x, embeddings)
    e_k = _gather_call()(emb_t, idx3.reshape(SC_NW, NCH, CH))
    return e_k.reshape(_inputs.shape), loss[0, 0]


# speedup vs baseline: 1.0605x; 1.0059x over previous
import functools

import jax
import jax.numpy as jnp
from jax import lax
from jax.experimental import pallas as pl
from jax.experimental.pallas import tpu as pltpu
from jax.experimental.pallas import tpu_sc as plsc

N_TOK = 16384
D = 256
K = 1024
BN = 4096
NB = N_TOK // BN

SC_NC = 2
SC_NS = 16
SC_NW = SC_NC * SC_NS
B_PER_W = N_TOK // SC_NW
CH = 128
NCH = B_PER_W // CH
NBUF = 2


def _dist_body(x_ref, emb_ref, idx_ref, loss_ref, embt_ref, emb2_s, e2_s):
    i = pl.program_id(0)

    @pl.when(i == 0)
    def _():
        emb = emb_ref[...]
        emb2_s[...] = emb + emb
        e2_s[...] = jnp.sum(emb * emb, axis=0, keepdims=True)
        embt_ref[...] = emb.T
        loss_ref[0, 0] = 0.0

    x = x_ref[...]
    cross2 = jnp.dot(x, emb2_s[...], preferred_element_type=jnp.float32)
    z2 = jnp.sum(x * x, axis=1, keepdims=True)
    e2 = e2_s[...]
    LC = 128
    z2b = jnp.broadcast_to(z2, (BN, LC))
    col0 = lax.broadcasted_iota(jnp.int32, (BN, LC), 1).astype(jnp.float32)
    rv = None
    ri = col0
    for t in range(K // LC):
        v = (z2b + e2[:, t * LC:(t + 1) * LC]) - cross2[:, t * LC:(t + 1) * LC]
        if t == 0:
            rv = v
        else:
            lt = v < rv
            ri = jnp.where(lt, col0 + float(t * LC), ri)
            rv = jnp.minimum(rv, v)
    rvT = rv.T
    riT = ri.T
    mT = jnp.min(rvT, axis=0, keepdims=True)
    idxf = jnp.min(jnp.where(rvT == mT, riT, float(K)), axis=0)
    idx_ref[0, 0, :] = idxf.astype(jnp.int32)

    loss_ref[0, 0] += jnp.sum(mT)

    @pl.when(i == NB - 1)
    def _():
        loss_ref[0, 0] *= 1.25 / (N_TOK * D)


_dist_call = pl.pallas_call(
    _dist_body,
    grid=(NB,),
    in_specs=[
        pl.BlockSpec((BN, D), lambda i: (i, 0)),
        pl.BlockSpec((D, K), lambda i: (0, 0)),
    ],
    out_specs=[
        pl.BlockSpec((1, 1, BN), lambda i: (i, 0, 0)),
        pl.BlockSpec((1, 1), lambda i: (0, 0), memory_space=pltpu.SMEM),
        pl.BlockSpec((K, D), lambda i: (0, 0)),
    ],
    out_shape=[
        jax.ShapeDtypeStruct((NB, 1, BN), jnp.int32),
        jax.ShapeDtypeStruct((1, 1), jnp.float32),
        jax.ShapeDtypeStruct((K, D), jnp.float32),
    ],
    scratch_shapes=[
        pltpu.VMEM((D, K), jnp.float32),
        pltpu.VMEM((1, K), jnp.float32),
    ],
    compiler_params=pltpu.CompilerParams(
        dimension_semantics=("arbitrary",),
    ),
)


def _gather_body(table_hbm, idx_hbm, out_hbm, idx_v, *rest):
    rows = rest[:NBUF]
    gsem = rest[NBUF:2 * NBUF]
    ssem = rest[2 * NBUF:]
    wid = lax.axis_index("s") * SC_NC + lax.axis_index("c")
    base = wid * B_PER_W
    pltpu.sync_copy(idx_hbm.at[wid], idx_v)
    gathers = [None] * NCH
    stores = [None] * NCH
    for c in range(NBUF):
        gathers[c] = pltpu.async_copy(
            table_hbm.at[idx_v.at[c]], rows[c], gsem[c])
    for c in range(NCH):
        b = c % NBUF
        gathers[c].wait()
        stores[c] = pltpu.async_copy(
            rows[b], out_hbm.at[pl.ds(base + c * CH, CH)], ssem[b])
        if c + NBUF < NCH:
            stores[c].wait()
            gathers[c + NBUF] = pltpu.async_copy(
                table_hbm.at[idx_v.at[c + NBUF]], rows[b], gsem[b])
    for c in range(NCH - NBUF, NCH):
        stores[c].wait()


@functools.cache
def _gather_call():
    return functools.partial(
        pl.kernel,
        out_type=jax.ShapeDtypeStruct((N_TOK, D), jnp.float32),
        mesh=plsc.VectorSubcoreMesh(
            core_axis_name="c", subcore_axis_name="s",
            num_cores=SC_NC, num_subcores=SC_NS,
        ),
        scratch_types=(
            [pltpu.VMEM((NCH, CH), jnp.int32)]
            + [pltpu.VMEM((CH, D), jnp.float32)] * NBUF
            + [pltpu.SemaphoreType.DMA] * (2 * NBUF)
        ),
    )(_gather_body)


def kernel(_inputs, embeddings):
    x = _inputs.reshape(N_TOK, D)
    idx3, loss, emb_t = _dist_call(x, embeddings)
    e_k = _gather_call()(emb_t, idx3.reshape(SC_NW, NCH, CH))
    return e_k.reshape(_inputs.shape), loss[0, 0]
